# Initial kernel scaffold; baseline (speedup 1.0000x reference)
import jax
import jax.numpy as jnp
from jax.experimental import pallas as pl


def kernel(node_features, ref_node_features, edge_features, ref_edge_features, atom_type, edge_type, onsite_weight, hopping_weight, mask_to_nrme, mask_to_erme):
    raise NotImplementedError("write your pallas kernel here")


# speedup vs baseline: 2.3140x; 2.3140x over previous
import functools

import jax
import jax.numpy as jnp
from jax import lax
from jax.experimental import pallas as pl
from jax.experimental.pallas import tpu as pltpu
from jax.experimental.pallas import tpu_sc as plsc

N_NODES = 10000
N_EDGES = 320000
D = 128
NT = 4
NBT = 16
NW = 32
EW = N_EDGES // NW
CHUNK = 80
N_WORKERS_NODES = 25
NODE_ROWS = N_NODES // N_WORKERS_NODES

_mesh = plsc.VectorSubcoreMesh(core_axis_name="c", subcore_axis_name="s")


def _zero_vmem(ref, n, zeros16):
    def body(i, carry):
        ref[pl.ds(i * 16, 16)] = zeros16
        return carry
    lax.fori_loop(0, n // 16, body, 0)


def _process_rows(nrows, ebuf, rbuf, tbuf, acc, cnt, ones16):
    def group_body(g, carry):
        tvec = tbuf[pl.ds(g * 16, 16)]
        for j in range(16):
            t = tvec[j]
            tb = t * 256
            row = g * 16 + j
            for b in range(8):
                fo = b * 16
                e = ebuf[row, pl.ds(fo, 16)]
                r = rbuf[row, pl.ds(fo, 16)]
                d = e - r
                acc[pl.ds(tb + fo, 16)] += jnp.abs(d)
                acc[pl.ds(tb + fo + 128, 16)] += d * d
            cnt[pl.ds(t * 16, 16)] += ones16
        return carry
    lax.fori_loop(0, nrows // 16, group_body, 0)


@functools.partial(
    pl.kernel,
    mesh=_mesh,
    out_type=(
        jax.ShapeDtypeStruct((NW, NBT * 2 * D), jnp.float32),
        jax.ShapeDtypeStruct((NW, NBT * 16), jnp.float32),
        jax.ShapeDtypeStruct((NW, NT * 2 * D), jnp.float32),
        jax.ShapeDtypeStruct((NW, NT * 16), jnp.float32),
    ),
    scratch_types=[
        pltpu.VMEM((CHUNK, D), jnp.float32),
        pltpu.VMEM((CHUNK, D), jnp.float32),
        pltpu.VMEM((CHUNK,), jnp.int32),
        pltpu.VMEM((NBT * 2 * D,), jnp.float32),
        pltpu.VMEM((NBT * 16,), jnp.float32),
        pltpu.VMEM((NT * 2 * D,), jnp.float32),
        pltpu.VMEM((NT * 16,), jnp.float32),
    ],
)
def _sc_accumulate(ef, rf, et, nf, rnf, at,
                   out_ep, out_ec, out_np, out_nc,
                   ebuf, rbuf, tbuf, acc_e, cnt_e, acc_n, cnt_n):
    wid = lax.axis_index("s") * 2 + lax.axis_index("c")
    ones16 = jnp.ones((16,), jnp.float32)
    zeros16 = jnp.zeros((16,), jnp.float32)

    _zero_vmem(acc_e, NBT * 2 * D, zeros16)
    _zero_vmem(acc_n, NT * 2 * D, zeros16)
    _zero_vmem(cnt_e, NBT * 16, zeros16)
    _zero_vmem(cnt_n, NT * 16, zeros16)

    estart = wid * EW

    def echunk(c, carry):
        off = estart + c * CHUNK
        pltpu.sync_copy(ef.at[pl.ds(off, CHUNK)], ebuf)
        pltpu.sync_copy(rf.at[pl.ds(off, CHUNK)], rbuf)
        pltpu.sync_copy(et.at[pl.ds(off, CHUNK)], tbuf)
        _process_rows(CHUNK, ebuf, rbuf, tbuf, acc_e, cnt_e, ones16)
        return carry

    lax.fori_loop(0, EW // CHUNK, echunk, 0)

    @pl.when(wid < N_WORKERS_NODES)
    def _():
        nstart = wid * NODE_ROWS

        def nchunk(c, carry):
            off = nstart + c * CHUNK
            pltpu.sync_copy(nf.at[pl.ds(off, CHUNK)], ebuf)
            pltpu.sync_copy(rnf.at[pl.ds(off, CHUNK)], rbuf)
            pltpu.sync_copy(at.at[pl.ds(off, CHUNK)], tbuf)
            _process_rows(CHUNK, ebuf, rbuf, tbuf, acc_n, cnt_n, ones16)
            return carry

        lax.fori_loop(0, NODE_ROWS // CHUNK, nchunk, 0)

    pltpu.sync_copy(acc_e, out_ep.at[wid])
    pltpu.sync_copy(cnt_e, out_ec.at[wid])
    pltpu.sync_copy(acc_n, out_np.at[wid])
    pltpu.sync_copy(cnt_n, out_nc.at[wid])


def _finish_body(epa, eps, ec, npa, nps, nc, ow, hw, nm, em, out):
    sabs_e = jnp.sum(epa[:], axis=0)
    ssq_e = jnp.sum(eps[:], axis=0)
    ce = jnp.sum(ec[:], axis=0)[:, :1]
    mean_abs_e = sabs_e / ce
    mean_sq_e = ssq_e / ce
    emf = em[:]
    den_e = jnp.sum(emf)
    h1 = jnp.sum(hw[:] * mean_abs_e * emf) / den_e
    h2 = jnp.sqrt(jnp.sum(hw[:] ** 2 * mean_sq_e * emf) / den_e)
    hop = 0.5 * (h1 + h2)

    sabs_n = jnp.sum(npa[:], axis=0)
    ssq_n = jnp.sum(nps[:], axis=0)
    cn = jnp.sum(nc[:], axis=0)[:, :1]
    mean_abs_n = sabs_n / cn
    mean_sq_n = ssq_n / cn
    nmf = nm[:]
    den_n = jnp.sum(nmf)
    o1 = jnp.sum(ow[:] * mean_abs_n * nmf) / den_n
    o2 = jnp.sqrt(jnp.sum(ow[:] ** 2 * mean_sq_n * nmf) / den_n)
    ons = 0.5 * (o1 + o2)

    out[:, :] = (0.5 * (ons + hop)).reshape(1, 1)


_finish = pl.pallas_call(
    _finish_body,
    out_shape=jax.ShapeDtypeStruct((1, 1), jnp.float32),
)


def kernel(node_features, ref_node_features, edge_features, ref_edge_features,
           atom_type, edge_type, onsite_weight, hopping_weight,
           mask_to_nrme, mask_to_erme):
    ep, ec, npart, nc = _sc_accumulate(
        edge_features, ref_edge_features, edge_type,
        node_features, ref_node_features, atom_type)
    ep4 = ep.reshape(NW, NBT, 2, D)
    np4 = npart.reshape(NW, NT, 2, D)
    out = _finish(ep4[:, :, 0, :], ep4[:, :, 1, :],
                  ec.reshape(NW, NBT, 16),
                  np4[:, :, 0, :], np4[:, :, 1, :],
                  nc.reshape(NW, NT, 16),
                  onsite_weight, hopping_weight,
                  mask_to_nrme.astype(jnp.float32),
                  mask_to_erme.astype(jnp.float32))
    return out.reshape(())


# speedup vs baseline: 4.3456x; 1.8779x over previous
import functools

import jax
import jax.numpy as jnp
from jax import lax
from jax.experimental import pallas as pl
from jax.experimental.pallas import tpu as pltpu
from jax.experimental.pallas import tpu_sc as plsc

N_NODES = 10000
N_EDGES = 320000
D = 128
NT = 4
NBT = 16
NW = 32
EW = N_EDGES // NW
CHUNK = 80
N_WORKERS_NODES = 25
NODE_ROWS = N_NODES // N_WORKERS_NODES
NACC = 4

_mesh = plsc.VectorSubcoreMesh(core_axis_name="c", subcore_axis_name="s")


def _zero_vmem(ref, n, zeros16):
    def body(i, carry):
        ref[pl.ds(i * 16, 16)] = zeros16
        return carry
    lax.fori_loop(0, n // 16, body, 0)


def _process_chunk(ebuf, rbuf, tbuf, aabs, asq, acnt, ones16):
    def quad_body(q, carry):
        tvec = tbuf[pl.ds(q * NACC, 16)]
        for j in range(NACC):
            t = tvec[j]
            tb = t * D
            row = q * NACC + j
            a_ab, a_sq, a_ct = aabs[j], asq[j], acnt[j]
            for b in range(8):
                fo = b * 16
                d = ebuf[row, pl.ds(fo, 16)] - rbuf[row, pl.ds(fo, 16)]
                plsc.addupdate(a_ab.at[pl.ds(tb + fo, 16)], jnp.abs(d))
                plsc.addupdate(a_sq.at[pl.ds(tb + fo, 16)], d * d)
            plsc.addupdate(a_ct.at[pl.ds(t * 16, 16)], ones16)
        return carry
    lax.fori_loop(0, CHUNK // NACC, quad_body, 0)


def _merge_into_0(refs, n):
    def body(i, carry):
        sl = pl.ds(i * 16, 16)
        v = refs[0][sl]
        for k in range(1, NACC):
            v = v + refs[k][sl]
        refs[0][sl] = v
        return carry
    lax.fori_loop(0, n // 16, body, 0)


_SCRATCH = (
    [pltpu.VMEM((CHUNK, D), jnp.float32) for _ in range(4)]
    + [pltpu.VMEM((CHUNK + 16,), jnp.int32) for _ in range(2)]
    + [pltpu.VMEM((NBT * D,), jnp.float32) for _ in range(2 * NACC)]
    + [pltpu.VMEM((NBT * 16,), jnp.float32) for _ in range(NACC)]
    + [pltpu.VMEM((NT * D,), jnp.float32) for _ in range(2 * NACC)]
    + [pltpu.VMEM((NT * 16,), jnp.float32) for _ in range(NACC)]
    + [pltpu.SemaphoreType.DMA, pltpu.SemaphoreType.DMA]
)


@functools.partial(
    pl.kernel,
    mesh=_mesh,
    out_type=(
        jax.ShapeDtypeStruct((NW, NBT * D), jnp.float32),
        jax.ShapeDtypeStruct((NW, NBT * D), jnp.float32),
        jax.ShapeDtypeStruct((NW, NBT * 16), jnp.float32),
        jax.ShapeDtypeStruct((NW, NT * D), jnp.float32),
        jax.ShapeDtypeStruct((NW, NT * D), jnp.float32),
        jax.ShapeDtypeStruct((NW, NT * 16), jnp.float32),
    ),
    scratch_types=_SCRATCH,
)
def _sc_accumulate(ef, rf, et, nf, rnf, at,
                   out_ea, out_es, out_ec, out_na, out_ns, out_nc,
                   e0, e1, r0, r1, t0, t1, *rest):
    aabs_e = rest[0:NACC]
    asq_e = rest[NACC:2 * NACC]
    acnt_e = rest[2 * NACC:3 * NACC]
    aabs_n = rest[3 * NACC:4 * NACC]
    asq_n = rest[4 * NACC:5 * NACC]
    acnt_n = rest[5 * NACC:6 * NACC]
    sem0, sem1 = rest[6 * NACC], rest[6 * NACC + 1]

    wid = lax.axis_index("s") * 2 + lax.axis_index("c")
    ones16 = jnp.ones((16,), jnp.float32)
    zeros16 = jnp.zeros((16,), jnp.float32)

    for k in range(NACC):
        _zero_vmem(aabs_e[k], NBT * D, zeros16)
        _zero_vmem(asq_e[k], NBT * D, zeros16)
        _zero_vmem(acnt_e[k], NBT * 16, zeros16)
        _zero_vmem(aabs_n[k], NT * D, zeros16)
        _zero_vmem(asq_n[k], NT * D, zeros16)
        _zero_vmem(acnt_n[k], NT * 16, zeros16)

    def start(feat, ref_feat, typ, base, c, eb, rb, tb, sem):
        off = base + c * CHUNK
        pltpu.async_copy(feat.at[pl.ds(off, CHUNK)], eb, sem)
        pltpu.async_copy(ref_feat.at[pl.ds(off, CHUNK)], rb, sem)
        pltpu.async_copy(typ.at[pl.ds(off, CHUNK)], tb.at[pl.ds(0, CHUNK)], sem)

    def wait(feat, ref_feat, typ, eb, rb, tb, sem):
        pltpu.make_async_copy(feat.at[pl.ds(0, CHUNK)], eb, sem).wait()
        pltpu.make_async_copy(ref_feat.at[pl.ds(0, CHUNK)], rb, sem).wait()
        pltpu.make_async_copy(typ.at[pl.ds(0, CHUNK)], tb.at[pl.ds(0, CHUNK)], sem).wait()

    def run_stream(feat, ref_feat, typ, base, nchunks, aabs, asq, acnt):
        start(feat, ref_feat, typ, base, 0, e0, r0, t0, sem0)

        def pair(k, carry):
            c0 = 2 * k
            start(feat, ref_feat, typ, base, c0 + 1, e1, r1, t1, sem1)
            wait(feat, ref_feat, typ, e0, r0, t0, sem0)
            _process_chunk(e0, r0, t0, aabs, asq, acnt, ones16)

            @pl.when(c0 + 2 < nchunks)
            def _():
                start(feat, ref_feat, typ, base, c0 + 2, e0, r0, t0, sem0)

            wait(feat, ref_feat, typ, e1, r1, t1, sem1)
            _process_chunk(e1, r1, t1, aabs, asq, acnt, ones16)
            return carry

        lax.fori_loop(0, (nchunks - 1) // 2, pair, 0)
        wait(feat, ref_feat, typ, e0, r0, t0, sem0)
        _process_chunk(e0, r0, t0, aabs, asq, acnt, ones16)

    run_stream(ef, rf, et, wid * EW, EW // CHUNK, aabs_e, asq_e, acnt_e)

    @pl.when(wid < N_WORKERS_NODES)
    def _():
        run_stream(nf, rnf, at, wid * NODE_ROWS, NODE_ROWS // CHUNK,
                   aabs_n, asq_n, acnt_n)

    _merge_into_0(aabs_e, NBT * D)
    _merge_into_0(asq_e, NBT * D)
    _merge_into_0(acnt_e, NBT * 16)
    _merge_into_0(aabs_n, NT * D)
    _merge_into_0(asq_n, NT * D)
    _merge_into_0(acnt_n, NT * 16)

    pltpu.sync_copy(aabs_e[0], out_ea.at[wid])
    pltpu.sync_copy(asq_e[0], out_es.at[wid])
    pltpu.sync_copy(acnt_e[0], out_ec.at[wid])
    pltpu.sync_copy(aabs_n[0], out_na.at[wid])
    pltpu.sync_copy(asq_n[0], out_ns.at[wid])
    pltpu.sync_copy(acnt_n[0], out_nc.at[wid])


def _finish_body(epa, eps, ec, npa, nps, nc, ow, hw, nm, em, out):
    sabs_e = jnp.sum(epa[:], axis=0)
    ssq_e = jnp.sum(eps[:], axis=0)
    ce = jnp.sum(ec[:], axis=0)[:, :1]
    mean_abs_e = sabs_e / ce
    mean_sq_e = ssq_e / ce
    emf = em[:]
    den_e = jnp.sum(emf)
    h1 = jnp.sum(hw[:] * mean_abs_e * emf) / den_e
    h2 = jnp.sqrt(jnp.sum(hw[:] ** 2 * mean_sq_e * emf) / den_e)
    hop = 0.5 * (h1 + h2)

    sabs_n = jnp.sum(npa[:], axis=0)
    ssq_n = jnp.sum(nps[:], axis=0)
    cn = jnp.sum(nc[:], axis=0)[:, :1]
    mean_abs_n = sabs_n / cn
    mean_sq_n = ssq_n / cn
    nmf = nm[:]
    den_n = jnp.sum(nmf)
    o1 = jnp.sum(ow[:] * mean_abs_n * nmf) / den_n
    o2 = jnp.sqrt(jnp.sum(ow[:] ** 2 * mean_sq_n * nmf) / den_n)
    ons = 0.5 * (o1 + o2)

    out[:, :] = (0.5 * (ons + hop)).reshape(1, 1)


_finish = pl.pallas_call(
    _finish_body,
    out_shape=jax.ShapeDtypeStruct((1, 1), jnp.float32),
)


def kernel(node_features, ref_node_features, edge_features, ref_edge_features,
           atom_type, edge_type, onsite_weight, hopping_weight,
           mask_to_nrme, mask_to_erme):
    ea, es, ec, na, ns, nc = _sc_accumulate(
        edge_features, ref_edge_features, edge_type,
        node_features, ref_node_features, atom_type)
    out = _finish(ea.reshape(NW, NBT, D), es.reshape(NW, NBT, D),
                  ec.reshape(NW, NBT, 16),
                  na.reshape(NW, NT, D), ns.reshape(NW, NT, D),
                  nc.reshape(NW, NT, 16),
                  onsite_weight, hopping_weight,
                  mask_to_nrme.astype(jnp.float32),
                  mask_to_erme.astype(jnp.float32))
    return out.reshape(())
